# Initial kernel scaffold; baseline (speedup 1.0000x reference)
import jax
import jax.numpy as jnp
from jax.experimental import pallas as pl


def kernel(cart, rs, inta, params, oc_params, nn_params, initpot, atom_index, local_species, neigh_species):
    raise NotImplementedError("write your pallas kernel here")


# speedup vs baseline: 5.4020x; 5.4020x over previous
import functools
import math

import numpy as np
import jax
import jax.numpy as jnp
from jax import lax
from jax.experimental import pallas as pl
from jax.experimental.pallas import tpu as pltpu
from jax.experimental.pallas import tpu_sc as plsc

CUTOFF = 4.5
NWAVE = 7
NTYPE = 4
N_ATOMS = 10000
N_EDGES = 320000

NP = 10240
EP = 327680
BE = 2048
BA = 1024
CH = 128
NW_SC = 32
DUMP = N_ATOMS

_BLOCKS = ((0, 1), (1, 3), (4, 9))

f32 = jnp.float32
i32 = jnp.int32


def _make_refidx():
    idx = np.zeros((84,), np.int64)
    p = 0
    for L in range(3):
        for d in range(7):
            for k in range(7 - d):
                t = 7 * k - k * (k - 1) // 2 + d
                idx[p] = L * 28 + t
                p += 1
    return idx


_REFIDX = _make_refidx()

@functools.cache
def _sc_mesh():
    return plsc.VectorSubcoreMesh(core_axis_name="c", subcore_axis_name="s")


def _sc_gather(table, idx, D):
    ep = idx.shape[0]
    per_w = ep // NW_SC

    @functools.partial(
        pl.kernel,
        out_type=jax.ShapeDtypeStruct((ep, D), f32),
        mesh=_sc_mesh(),
        scratch_types=[
            pltpu.VMEM((CH,), i32),
            pltpu.VMEM((CH, D), f32),
            pltpu.SemaphoreType.DMA,
        ],
    )
    def k(table_hbm, idx_hbm, out_hbm, idx_v, rows_v, sem):
        wid = lax.axis_index("s") * 2 + lax.axis_index("c")
        base = wid * per_w

        @pl.loop(0, per_w, step=CH)
        def _(off):
            pltpu.sync_copy(idx_hbm.at[pl.ds(base + off, CH)], idx_v)
            pltpu.async_copy(table_hbm.at[idx_v], rows_v, sem).wait()
            pltpu.sync_copy(rows_v, out_hbm.at[pl.ds(base + off, CH)])

    return k(table, idx)


def _sc_scatter_add(vals, idx, D):
    ep = vals.shape[0]
    per_w = ep // NW_SC
    rows_per_sub = NP // 16
    zeros_chunk = jnp.zeros((CH, D), f32)

    @functools.partial(
        pl.kernel,
        out_type=jax.ShapeDtypeStruct((2, NP, D), f32),
        mesh=_sc_mesh(),
        scratch_types=[
            pltpu.VMEM((CH,), i32),
            pltpu.VMEM((CH, D), f32),
            pltpu.VMEM_SHARED((NP, D), f32),
            pltpu.SemaphoreType.DMA,
        ],
    )
    def k(vals_hbm, idx_hbm, zeros_hbm, out_hbm, idx_v, vals_v, acc_sh, sem):
        cid = lax.axis_index("c")
        sid = lax.axis_index("s")
        wid = sid * 2 + cid
        row0 = sid * rows_per_sub

        @pl.loop(0, rows_per_sub, step=CH)
        def _(r0):
            pltpu.sync_copy(zeros_hbm, acc_sh.at[pl.ds(row0 + r0, CH)])

        plsc.subcore_barrier()

        base = wid * per_w

        @pl.loop(0, per_w, step=CH)
        def _(off):
            pltpu.sync_copy(idx_hbm.at[pl.ds(base + off, CH)], idx_v)
            pltpu.sync_copy(vals_hbm.at[pl.ds(base + off, CH)], vals_v)
            pltpu.sync_copy(vals_v, acc_sh.at[idx_v], add=True)

        plsc.subcore_barrier()

        @pl.loop(0, rows_per_sub, step=CH)
        def _(r0):
            pltpu.sync_copy(acc_sh.at[pl.ds(row0 + r0, CH)],
                            out_hbm.at[cid].at[pl.ds(row0 + r0, CH)])

    return k(vals, idx, zeros_chunk)


def _shl(x, d, width):
    if d == 0:
        return x
    return jnp.concatenate([x[:, d:width], jnp.zeros((x.shape[0], d), f32)], axis=1)


def _shr(x, d, width):
    if d == 0:
        return x
    return jnp.concatenate([jnp.zeros((x.shape[0], d), f32), x[:, : width - d]], axis=1)


def _species_masks(sp):
    return [(sp == t).astype(f32) for t in range(NTYPE)]


def _sel(masks, ref):
    v = ref[...]
    out = masks[0] * v[0:1, :]
    for t in range(1, NTYPE):
        out = out + masks[t] * v[t:t + 1, :]
    return out


def _mdot(masks, x, w_ref):
    out = None
    for t in range(NTYPE):
        ht = jnp.dot(x, w_ref[t], preferred_element_type=f32) * masks[t]
        out = ht if out is None else out + ht
    return out


def _silu_grad(v, sig):
    return sig * (1.0 + v * (1.0 - sig))


def _mlp_core(x, masks, W0, b0, g0, a0, W1, b1, a1,
              Wout=None, bout=None, W0T=None, W1T=None, WoutT=None,
              ybar=None, ybar_is_ones=False, need_y=True):
    h0 = _mdot(masks, x, W0) + _sel(masks, b0)
    mu = jnp.mean(h0, axis=1, keepdims=True)
    xc = h0 - mu
    var = jnp.mean(xc * xc, axis=1, keepdims=True)
    inv = lax.rsqrt(var + 1e-5)
    xh = xc * inv
    g0s = _sel(masks, g0)
    a0s = _sel(masks, a0)
    a1s = _sel(masks, a1)
    ln = g0s * xh
    sig_ln = jax.nn.sigmoid(ln)
    h1 = a0s * ln * sig_ln
    z = _mdot(masks, h1, W1) + _sel(masks, b1)
    sig_z = jax.nn.sigmoid(z)
    y = None
    if need_y:
        h2 = h1 + a1s * z * sig_z
        y = _mdot(masks, h2, Wout) + _sel(masks, bout)
    if ybar is None:
        return y, None
    if ybar_is_ones:
        v = WoutT[...]
        h2b = masks[0] * v[0]
        for t in range(1, NTYPE):
            h2b = h2b + masks[t] * v[t]
    else:
        h2b = _mdot(masks, ybar, WoutT)
    zb = h2b * a1s * _silu_grad(z, sig_z)
    h1b = h2b + _mdot(masks, zb, W1T)
    lnb = h1b * a0s * _silu_grad(ln, sig_ln)
    xhb = lnb * g0s
    h0b = inv * (xhb - jnp.mean(xhb, axis=1, keepdims=True)
                 - xh * jnp.mean(xhb * xh, axis=1, keepdims=True))
    xb = _mdot(masks, h0b, W0T)
    return y, xb


def _geom_body(cc_ref, cn_ref, sp_ref, rs_ref, inta_ref, g_ref):
    cc = cc_ref[...]
    cn = cn_ref[...]
    dvec = cn[:, 0:3] - cc[:, 0:3]
    r = jnp.sqrt(jnp.sum(dvec * dvec, axis=1, keepdims=True) + 1e-12)
    sp = sp_ref[...]
    rsv = rs_ref[...]
    intav = inta_ref[...]
    a = jnp.zeros((BE, 8), f32)
    rho = jnp.zeros((BE, 8), f32)
    for t in range(NTYPE):
        m = (sp == t).astype(f32)
        a = a + m * intav[t:t + 1, :]
        rho = rho + m * rsv[t:t + 1, :]
    ex = jnp.exp(-a * jnp.square(r - rho))
    q = 0.5 * jnp.cos((math.pi / CUTOFF) * r) + 0.5
    radial = ex * (q * q)
    g_ref[...] = jnp.concatenate(
        [dvec, r, radial[:, 0:7], jnp.zeros((BE, 5), f32)], axis=1)


def _orb_body(g_ref, cn_ref, o_ref):
    g = g_ref[...]
    dvec = g[:, 0:3]
    r = g[:, 3:4]
    radial = g[:, 4:11]
    u = dvec / r
    w = cn_ref[...][:, 0:7] * radial
    pieces = [w]
    for aa in range(3):
        pieces.append(u[:, aa:aa + 1] * w)
    for aa in range(3):
        for bb in range(3):
            pieces.append(u[:, aa:aa + 1] * u[:, bb:bb + 1] * w)
    o_ref[...] = jnp.concatenate(pieces + [jnp.zeros((BE, 37), f32)], axis=1)


def _gram_fwd_body(p_ref, den_ref, sum_ref):
    s = (p_ref[0] + p_ref[1])[:, 0:96]
    sum_ref[...] = s
    P = []
    for d in range(7):
        P.append(s * _shl(s, d, 96))
    cols = []
    for L, (j0, nc) in enumerate(_BLOCKS):
        for d in range(7):
            acc = None
            for j in range(j0, j0 + nc):
                sl = P[d][:, 7 * j:7 * j + 7]
                acc = sl if acc is None else acc + sl
            cols.append(acc[:, 0:7 - d])
    den_ref[...] = jnp.concatenate(cols, axis=1)


def _gram_bwd_body(db_ref, sum_ref, sb_ref):
    db = db_ref[...]
    s = sum_ref[...]
    out = jnp.zeros((BA, 96), f32)
    for d in range(7):
        pieces = []
        for L in range(3):
            off = L * 28 + 7 * d - d * (d - 1) // 2
            sl = db[:, off:off + 7 - d]
            if d:
                sl = jnp.concatenate([sl, jnp.zeros((BA, d), f32)], axis=1)
            pieces += [sl] * _BLOCKS[L][1]
        A = jnp.concatenate(pieces + [jnp.zeros((BA, 5), f32)], axis=1)
        if d == 0:
            out = out + 2.0 * A * s
        else:
            out = out + A * _shl(s, d, 96) + _shr(A * s, d, 96)
    sb_ref[...] = jnp.concatenate([out, jnp.zeros((BA, 32), f32)], axis=1)


def _c0_body(sp_ref, params_ref, out_ref):
    masks = _species_masks(sp_ref[...])
    c0 = _sel(masks, params_ref)
    out_ref[...] = jnp.concatenate([c0[:, 0:7], jnp.zeros((BA, 121), f32)], axis=1)


def _oc_fwd_body(den_ref, sp_ref, params_ref, W0, b0, g0, a0, W1, b1, a1,
                 Wout, bout, out_ref):
    masks = _species_masks(sp_ref[...])
    y, _ = _mlp_core(den_ref[...], masks, W0, b0, g0, a0, W1, b1, a1,
                     Wout=Wout, bout=bout)
    c0 = _sel(masks, params_ref)
    out_ref[...] = jnp.concatenate(
        [c0[:, 0:7] + y, jnp.zeros((BA, 121), f32)], axis=1)


def _oc_bwd_body(den_ref, sp_ref, cb_ref, W0, b0, g0, a0, W1, b1, a1,
                 W0T, W1T, WoutT, db_ref):
    masks = _species_masks(sp_ref[...])
    ybar = (cb_ref[0] + cb_ref[1])[:, 0:7]
    _, xb = _mlp_core(den_ref[...], masks, W0, b0, g0, a0, W1, b1, a1,
                      W0T=W0T, W1T=W1T, WoutT=WoutT, ybar=ybar, need_y=False)
    db_ref[...] = xb


def _nn_body(den_ref, sp_ref, init_ref, W0, b0, g0, a0, W1, b1, a1,
             Wout, bout, W0T, W1T, WoutT, out_ref, vsum_ref, db_ref):
    masks = _species_masks(sp_ref[...])
    y, xb = _mlp_core(den_ref[...], masks, W0, b0, g0, a0, W1, b1, a1,
                      Wout=Wout, bout=bout, W0T=W0T, W1T=W1T, WoutT=WoutT,
                      ybar=True, ybar_is_ones=True)
    o = y + init_ref[...]
    out_ref[...] = o
    db_ref[...] = xb
    i = pl.program_id(0)
    rid = lax.broadcasted_iota(i32, (BA, 1), 0) + i * BA
    m = (rid < N_ATOMS).astype(f32)

    @pl.when(i == 0)
    def _():
        vsum_ref[...] = jnp.zeros((1, 1), f32)

    vsum_ref[...] += jnp.sum(o * m, keepdims=True)


def _edge_bwd_body_base(g_ref, cn_ref, sg_ref, sp_ref, rs_ref, inta_ref,
                        dv2_ref, dv_ref, cb_ref):
    g = g_ref[...]
    dvec = g[:, 0:3]
    r = g[:, 3:4]
    radial = g[:, 4:11]
    sg = sg_ref[...]
    cnv = cn_ref[...][:, 0:7]
    inv_r = 1.0 / r
    u = dvec * inv_r
    w = cnv * radial

    coefs = [None]
    for aa in range(3):
        coefs.append(u[:, aa:aa + 1])
    for aa in range(3):
        for bb in range(3):
            coefs.append(u[:, aa:aa + 1] * u[:, bb:bb + 1])

    wbar = sg[:, 0:7]
    abar = [None]
    for j in range(1, 13):
        sl = sg[:, 7 * j:7 * j + 7]
        wbar = wbar + coefs[j] * sl
        abar.append(jnp.sum(sl * w, axis=1, keepdims=True))

    cb = wbar * radial
    radial_bar = wbar * cnv

    sp = sp_ref[...]
    rsv = rs_ref[...]
    intav = inta_ref[...]
    a = jnp.zeros((BE, 8), f32)
    rho = jnp.zeros((BE, 8), f32)
    for t in range(NTYPE):
        m = (sp == t).astype(f32)
        a = a + m * intav[t:t + 1, :]
        rho = rho + m * rsv[t:t + 1, :]
    a7 = a[:, 0:7]
    rho7 = rho[:, 0:7]
    ex = jnp.exp(-a7 * jnp.square(r - rho7))
    q = 0.5 * jnp.cos((math.pi / CUTOFF) * r) + 0.5
    qp = -(math.pi / (2.0 * CUTOFF)) * jnp.sin((math.pi / CUTOFF) * r)
    drad = ex * (-2.0 * a7 * (r - rho7)) * (q * q) + ex * 2.0 * q * qp
    rbar = jnp.sum(radial_bar * drad, axis=1, keepdims=True)

    ubars = []
    for aa in range(3):
        ub = abar[1 + aa]
        for bb in range(3):
            ub = ub + (abar[4 + 3 * aa + bb] + abar[4 + 3 * bb + aa]) * u[:, bb:bb + 1]
        ubars.append(ub)
    ubar = jnp.concatenate(ubars, axis=1)
    udot = jnp.sum(ubar * u, axis=1, keepdims=True)
    dv = (ubar - udot * u) * inv_r + rbar * u
    if dv2_ref is not None:
        dv = dv + dv2_ref[...][:, 0:3]
    dv_ref[...] = jnp.concatenate([dv, jnp.zeros((BE, 125), f32)], axis=1)
    cb_ref[...] = jnp.concatenate([cb, jnp.zeros((BE, 121), f32)], axis=1)


def _edge_bwd_body(g_ref, cn_ref, sg_ref, sp_ref, rs_ref, inta_ref,
                   dv_ref, cb_ref):
    _edge_bwd_body_base(g_ref, cn_ref, sg_ref, sp_ref, rs_ref, inta_ref,
                        None, dv_ref, cb_ref)


def _edge_bwd_add_body(g_ref, cn_ref, sg_ref, sp_ref, rs_ref, inta_ref,
                       dv2_ref, dv_ref, cb_ref):
    _edge_bwd_body_base(g_ref, cn_ref, sg_ref, sp_ref, rs_ref, inta_ref,
                        dv2_ref, dv_ref, cb_ref)


def _forces_body(pn_ref, pc_ref, f_ref):
    f_ref[...] = ((pc_ref[0] + pc_ref[1]) - (pn_ref[0] + pn_ref[1]))[:, 0:16]


def _espec(D):
    return pl.BlockSpec((BE, D), lambda i: (i, 0))


def _aspec(D):
    return pl.BlockSpec((BA, D), lambda i: (i, 0))


def _pspec(D):
    return pl.BlockSpec((2, BA, D), lambda i: (0, i, 0))


def _fullspec(arr):
    nd = arr.ndim
    return pl.BlockSpec(arr.shape, lambda i, _n=nd: (0,) * _n)


_EGRID = (EP // BE,)
_AGRID = (NP // BA,)


def _tc_geom(cc, cn, nsp, rs8, inta8):
    return pl.pallas_call(
        _geom_body, grid=_EGRID,
        in_specs=[_espec(128), _espec(128), _espec(1), _fullspec(rs8), _fullspec(inta8)],
        out_specs=_espec(16),
        out_shape=jax.ShapeDtypeStruct((EP, 16), f32),
    )(cc, cn, nsp, rs8, inta8)


def _tc_orb(G, Cn):
    return pl.pallas_call(
        _orb_body, grid=_EGRID,
        in_specs=[_espec(16), _espec(128)],
        out_specs=_espec(128),
        out_shape=jax.ShapeDtypeStruct((EP, 128), f32),
    )(G, Cn)


def _tc_gram_fwd(parts):
    return pl.pallas_call(
        _gram_fwd_body, grid=_AGRID,
        in_specs=[_pspec(128)],
        out_specs=[_aspec(84), _aspec(96)],
        out_shape=[jax.ShapeDtypeStruct((NP, 84), f32),
                   jax.ShapeDtypeStruct((NP, 96), f32)],
    )(parts)


def _tc_gram_bwd(db, summed):
    return pl.pallas_call(
        _gram_bwd_body, grid=_AGRID,
        in_specs=[_aspec(84), _aspec(96)],
        out_specs=_aspec(128),
        out_shape=jax.ShapeDtypeStruct((NP, 128), f32),
    )(db, summed)


def _tc_c0(sp, params8):
    return pl.pallas_call(
        _c0_body, grid=_AGRID,
        in_specs=[_aspec(1), _fullspec(params8)],
        out_specs=_aspec(128),
        out_shape=jax.ShapeDtypeStruct((NP, 128), f32),
    )(sp, params8)


def _tc_oc_fwd(den, sp, params8, p):
    args = (den, sp, params8, p["W0"], p["b0"], p["g0"], p["a0"],
            p["W1"], p["b1"], p["a1"], p["Wout"], p["bout"])
    return pl.pallas_call(
        _oc_fwd_body, grid=_AGRID,
        in_specs=[_aspec(84), _aspec(1)] + [_fullspec(a) for a in args[2:]],
        out_specs=_aspec(128),
        out_shape=jax.ShapeDtypeStruct((NP, 128), f32),
    )(*args)


def _tc_oc_bwd(den, sp, cbparts, p):
    args = (den, sp, cbparts, p["W0"], p["b0"], p["g0"], p["a0"],
            p["W1"], p["b1"], p["a1"], p["W0T"], p["W1T"], p["WoutT"])
    return pl.pallas_call(
        _oc_bwd_body, grid=_AGRID,
        in_specs=[_aspec(84), _aspec(1), _pspec(128)] + [_fullspec(a) for a in args[3:]],
        out_specs=_aspec(84),
        out_shape=jax.ShapeDtypeStruct((NP, 84), f32),
    )(*args)


def _tc_nn(den, sp, init11, p):
    args = (den, sp, init11, p["W0"], p["b0"], p["g0"], p["a0"],
            p["W1"], p["b1"], p["a1"], p["Wout"], p["bout"],
            p["W0T"], p["W1T"], p["WoutT"])
    return pl.pallas_call(
        _nn_body, grid=_AGRID,
        in_specs=[_aspec(84), _aspec(1), _fullspec(init11)]
                 + [_fullspec(a) for a in args[3:]],
        out_specs=[_aspec(1), pl.BlockSpec((1, 1), lambda i: (0, 0)), _aspec(84)],
        out_shape=[jax.ShapeDtypeStruct((NP, 1), f32),
                   jax.ShapeDtypeStruct((1, 1), f32),
                   jax.ShapeDtypeStruct((NP, 84), f32)],
    )(*args)


def _tc_edge_bwd(G, Cn, sg, nsp, rs8, inta8, dv2=None):
    outs = [jax.ShapeDtypeStruct((EP, 128), f32), jax.ShapeDtypeStruct((EP, 128), f32)]
    ospecs = [_espec(128), _espec(128)]
    if dv2 is None:
        return pl.pallas_call(
            _edge_bwd_body, grid=_EGRID,
            in_specs=[_espec(16), _espec(128), _espec(128), _espec(1),
                      _fullspec(rs8), _fullspec(inta8)],
            out_specs=ospecs, out_shape=outs,
        )(G, Cn, sg, nsp, rs8, inta8)
    return pl.pallas_call(
        _edge_bwd_add_body, grid=_EGRID,
        in_specs=[_espec(16), _espec(128), _espec(128), _espec(1),
                  _fullspec(rs8), _fullspec(inta8), _espec(128)],
        out_specs=ospecs, out_shape=outs,
    )(G, Cn, sg, nsp, rs8, inta8, dv2)


def _tc_forces(pn, pc):
    return pl.pallas_call(
        _forces_body, grid=_AGRID,
        in_specs=[_pspec(128), _pspec(128)],
        out_specs=_aspec(16),
        out_shape=jax.ShapeDtypeStruct((NP, 16), f32),
    )(pn, pc)


def _prep_mlp(p, nout):
    W0 = p["W0"][:, _REFIDX, :]
    out = {
        "W0": W0,
        "b0": p["b0"], "g0": p["g0"], "a0": p["a0"],
        "W1": p["W1"], "b1": p["b1"], "a1": p["a1"],
        "Wout": p["Wout"], "bout": p["bout"],
        "W0T": jnp.swapaxes(W0, 1, 2),
        "W1T": jnp.swapaxes(p["W1"], 1, 2),
        "WoutT": jnp.swapaxes(p["Wout"], 1, 2),
    }
    return out


def kernel(cart, rs, inta, params, oc_params, nn_params, initpot,
           atom_index, local_species, neigh_species):
    center = atom_index[0].astype(i32)
    neigh = atom_index[1].astype(i32)
    pad_idx = jnp.full((EP - N_EDGES,), DUMP, i32)
    center_p = jnp.concatenate([center, pad_idx])
    neigh_p = jnp.concatenate([neigh, pad_idx])
    nsp_p = jnp.concatenate(
        [neigh_species.astype(i32), jnp.zeros((EP - N_EDGES,), i32)]).reshape(EP, 1)
    sp_p = jnp.concatenate(
        [local_species.astype(i32), jnp.zeros((NP - N_ATOMS,), i32)]).reshape(NP, 1)
    cart128 = jnp.zeros((NP, 128), f32).at[:N_ATOMS, 0:3].set(cart)
    rs8 = jnp.zeros((NTYPE, 8), f32).at[:, 0:7].set(rs)
    inta8 = jnp.zeros((NTYPE, 8), f32).at[:, 0:7].set(inta)
    params8 = jnp.zeros((NTYPE, 8), f32).at[:, 0:7].set(params)
    ocP = _prep_mlp(oc_params, NWAVE)
    nnP = _prep_mlp(nn_params, 1)
    init11 = jnp.reshape(initpot, (1, 1)).astype(f32)

    cc = _sc_gather(cart128, center_p, 128)
    cn = _sc_gather(cart128, neigh_p, 128)
    G = _tc_geom(cc, cn, nsp_p, rs8, inta8)

    C0p = _tc_c0(sp_p, params8)
    Cn1 = _sc_gather(C0p, neigh_p, 128)
    orb1 = _tc_orb(G, Cn1)
    parts1 = _sc_scatter_add(orb1, center_p, 128)
    den1, sum1 = _tc_gram_fwd(parts1)

    C1p = _tc_oc_fwd(den1, sp_p, params8, ocP)
    Cn2 = _sc_gather(C1p, neigh_p, 128)
    orb2 = _tc_orb(G, Cn2)
    parts2 = _sc_scatter_add(orb2, center_p, 128)
    den2, sum2 = _tc_gram_fwd(parts2)

    out_full, vsum, dbar2 = _tc_nn(den2, sp_p, init11, nnP)

    sbar2 = _tc_gram_bwd(dbar2, sum2)
    sg2 = _sc_gather(sbar2, center_p, 128)
    dv2, cb2 = _tc_edge_bwd(G, Cn2, sg2, nsp_p, rs8, inta8)
    cbparts = _sc_scatter_add(cb2, neigh_p, 128)

    dbar1 = _tc_oc_bwd(den1, sp_p, cbparts, ocP)

    sbar1 = _tc_gram_bwd(dbar1, sum1)
    sg1 = _sc_gather(sbar1, center_p, 128)
    dvt, _ = _tc_edge_bwd(G, Cn1, sg1, nsp_p, rs8, inta8, dv2=dv2)

    pn = _sc_scatter_add(dvt, neigh_p, 128)
    pc = _sc_scatter_add(dvt, center_p, 128)
    F = _tc_forces(pn, pc)

    varene = vsum[0, 0]
    forces = F[:N_ATOMS, 0:3].reshape(-1)
    output = out_full[:N_ATOMS]
    return varene, forces, output


# speedup vs baseline: 13.9911x; 2.5900x over previous
import functools
import math

import numpy as np
import jax
import jax.numpy as jnp
from jax import lax
from jax.experimental import pallas as pl
from jax.experimental.pallas import tpu as pltpu
from jax.experimental.pallas import tpu_sc as plsc

CUTOFF = 4.5
NWAVE = 7
NTYPE = 4
N_ATOMS = 10000
N_EDGES = 320000

NP = 10240
EP = 327680
BE = 2048
BA = 1024
CH = 128
NW_SC = 32
DUMP = N_ATOMS

_BLOCKS = ((0, 1), (1, 3), (4, 9))

f32 = jnp.float32
i32 = jnp.int32


def _make_refidx():
    idx = np.zeros((84,), np.int64)
    p = 0
    for L in range(3):
        for d in range(7):
            for k in range(7 - d):
                t = 7 * k - k * (k - 1) // 2 + d
                idx[p] = L * 28 + t
                p += 1
    return idx


_REFIDX = _make_refidx()

@functools.cache
def _sc_mesh():
    return plsc.VectorSubcoreMesh(core_axis_name="c", subcore_axis_name="s")


def _sc_gather(table, idx, D):
    ep = idx.shape[0]
    per_w = ep // NW_SC

    @functools.partial(
        pl.kernel,
        out_type=jax.ShapeDtypeStruct((ep, D), f32),
        mesh=_sc_mesh(),
        scratch_types=[
            pltpu.VMEM((CH,), i32),
            pltpu.VMEM((CH, D), f32),
            pltpu.SemaphoreType.DMA,
        ],
    )
    def k(table_hbm, idx_hbm, out_hbm, idx_v, rows_v, sem):
        wid = lax.axis_index("s") * 2 + lax.axis_index("c")
        base = wid * per_w

        @pl.loop(0, per_w, step=CH)
        def _(off):
            pltpu.sync_copy(idx_hbm.at[pl.ds(base + off, CH)], idx_v)
            pltpu.async_copy(table_hbm.at[idx_v], rows_v, sem).wait()
            pltpu.sync_copy(rows_v, out_hbm.at[pl.ds(base + off, CH)])

    return k(table, idx)


def _sc_scatter_add(vals, idx, D):
    ep = vals.shape[0]
    per_w = ep // NW_SC
    rows_per_sub = NP // 16
    zeros_chunk = jnp.zeros((CH, D), f32)

    @functools.partial(
        pl.kernel,
        out_type=jax.ShapeDtypeStruct((2, NP, D), f32),
        mesh=_sc_mesh(),
        scratch_types=[
            pltpu.VMEM((CH,), i32),
            pltpu.VMEM((CH, D), f32),
            pltpu.VMEM_SHARED((NP, D), f32),
            pltpu.SemaphoreType.DMA,
        ],
    )
    def k(vals_hbm, idx_hbm, zeros_hbm, out_hbm, idx_v, vals_v, acc_sh, sem):
        cid = lax.axis_index("c")
        sid = lax.axis_index("s")
        wid = sid * 2 + cid
        row0 = sid * rows_per_sub

        @pl.loop(0, rows_per_sub, step=CH)
        def _(r0):
            pltpu.sync_copy(zeros_hbm, acc_sh.at[pl.ds(row0 + r0, CH)])

        plsc.subcore_barrier()

        base = wid * per_w

        @pl.loop(0, per_w, step=CH)
        def _(off):
            pltpu.sync_copy(idx_hbm.at[pl.ds(base + off, CH)], idx_v)
            pltpu.sync_copy(vals_hbm.at[pl.ds(base + off, CH)], vals_v)
            pltpu.sync_copy(vals_v, acc_sh.at[idx_v], add=True)

        plsc.subcore_barrier()

        @pl.loop(0, rows_per_sub, step=CH)
        def _(r0):
            pltpu.sync_copy(acc_sh.at[pl.ds(row0 + r0, CH)],
                            out_hbm.at[cid].at[pl.ds(row0 + r0, CH)])

    return k(vals, idx, zeros_chunk)


def _shl(x, d, width):
    if d == 0:
        return x
    return jnp.concatenate([x[:, d:width], jnp.zeros((x.shape[0], d), f32)], axis=1)


def _shr(x, d, width):
    if d == 0:
        return x
    return jnp.concatenate([jnp.zeros((x.shape[0], d), f32), x[:, : width - d]], axis=1)


def _species_masks(sp):
    return [(sp == t).astype(f32) for t in range(NTYPE)]


def _sel(masks, ref):
    v = ref[...]
    out = masks[0] * v[0:1, :]
    for t in range(1, NTYPE):
        out = out + masks[t] * v[t:t + 1, :]
    return out


def _mdot(masks, x, w_ref):
    out = None
    for t in range(NTYPE):
        ht = jnp.dot(x, w_ref[t], preferred_element_type=f32) * masks[t]
        out = ht if out is None else out + ht
    return out


def _silu_grad(v, sig):
    return sig * (1.0 + v * (1.0 - sig))


def _mlp_core(x, masks, W0, b0, g0, a0, W1, b1, a1,
              Wout=None, bout=None, W0T=None, W1T=None, WoutT=None,
              ybar=None, ybar_is_ones=False, need_y=True):
    h0 = _mdot(masks, x, W0) + _sel(masks, b0)
    mu = jnp.mean(h0, axis=1, keepdims=True)
    xc = h0 - mu
    var = jnp.mean(xc * xc, axis=1, keepdims=True)
    inv = lax.rsqrt(var + 1e-5)
    xh = xc * inv
    g0s = _sel(masks, g0)
    a0s = _sel(masks, a0)
    a1s = _sel(masks, a1)
    ln = g0s * xh
    sig_ln = jax.nn.sigmoid(ln)
    h1 = a0s * ln * sig_ln
    z = _mdot(masks, h1, W1) + _sel(masks, b1)
    sig_z = jax.nn.sigmoid(z)
    y = None
    if need_y:
        h2 = h1 + a1s * z * sig_z
        y = _mdot(masks, h2, Wout) + _sel(masks, bout)
    if ybar is None:
        return y, None
    if ybar_is_ones:
        v = WoutT[...]
        h2b = masks[0] * v[0]
        for t in range(1, NTYPE):
            h2b = h2b + masks[t] * v[t]
    else:
        h2b = _mdot(masks, ybar, WoutT)
    zb = h2b * a1s * _silu_grad(z, sig_z)
    h1b = h2b + _mdot(masks, zb, W1T)
    lnb = h1b * a0s * _silu_grad(ln, sig_ln)
    xhb = lnb * g0s
    h0b = inv * (xhb - jnp.mean(xhb, axis=1, keepdims=True)
                 - xh * jnp.mean(xhb * xh, axis=1, keepdims=True))
    xb = _mdot(masks, h0b, W0T)
    return y, xb


def _species_rows(spT, tab_ref):
    tabT = jnp.transpose(tab_ref[...])
    out = jnp.zeros((8, spT.shape[1]), f32)
    for t in range(NTYPE):
        m = (spT == float(t)).astype(f32)
        out = out + tabT[:, t:t + 1] * m
    return out


def _orb_from(u, radial, cT):
    w = cT * radial
    urows = [u[aa:aa + 1] for aa in range(3)]
    pieces = [w]
    for aa in range(3):
        pieces.append(urows[aa] * w)
    for aa in range(3):
        for bb in range(3):
            pieces.append((urows[aa] * urows[bb]) * w)
    oT = jnp.concatenate(pieces + [jnp.zeros((5, BE), f32)], axis=0)
    return jnp.concatenate(
        [jnp.transpose(oT), jnp.zeros((BE, 32), f32)], axis=1)


def _geom_orb1_body(cc_ref, cn_ref, sp_ref, rs_ref, inta_ref, par_ref,
                    gt_ref, o_ref):
    ccT = jnp.transpose(cc_ref[...][:, 0:8])
    cnT = jnp.transpose(cn_ref[...][:, 0:8])
    dvecT = cnT[0:3] - ccT[0:3]
    r2 = jnp.sum(dvecT * dvecT, axis=0, keepdims=True) + 1e-12
    r = jnp.sqrt(r2)
    inv_r = 1.0 / r
    spT = jnp.transpose(sp_ref[...].astype(f32))
    a7 = _species_rows(spT, inta_ref)[0:7]
    rho7 = _species_rows(spT, rs_ref)[0:7]
    ex = jnp.exp(-a7 * jnp.square(r - rho7))
    q = 0.5 * jnp.cos((math.pi / CUTOFF) * r) + 0.5
    qp = -(math.pi / (2.0 * CUTOFF)) * jnp.sin((math.pi / CUTOFF) * r)
    q2 = q * q
    radial = ex * q2
    drad = ex * (-2.0 * a7 * (r - rho7)) * q2 + ex * 2.0 * q * qp
    gt_ref[...] = jnp.concatenate(
        [dvecT, r, inv_r, radial, drad, jnp.zeros((13, BE), f32)], axis=0)
    u = dvecT * inv_r
    cT = _species_rows(spT, par_ref)[0:7]
    o_ref[...] = _orb_from(u, radial, cT)


def _orb_body(gt_ref, cn_ref, o_ref):
    gt = gt_ref[...]
    u = gt[0:3] * gt[4:5]
    radial = gt[5:12]
    cnT = jnp.transpose(cn_ref[...][:, 0:8])
    o_ref[...] = _orb_from(u, radial, cnT[0:7])


def _gram_fwd_body(p_ref, den_ref, sum_ref):
    s = (p_ref[0] + p_ref[1])[:, 0:96]
    sum_ref[...] = s
    P = []
    for d in range(7):
        P.append(s * _shl(s, d, 96))
    cols = []
    for L, (j0, nc) in enumerate(_BLOCKS):
        for d in range(7):
            acc = None
            for j in range(j0, j0 + nc):
                sl = P[d][:, 7 * j:7 * j + 7]
                acc = sl if acc is None else acc + sl
            cols.append(acc[:, 0:7 - d])
    den_ref[...] = jnp.concatenate(cols, axis=1)


def _gram_bwd_body(db_ref, sum_ref, sb_ref):
    db = db_ref[...]
    s = sum_ref[...]
    out = jnp.zeros((BA, 96), f32)
    for d in range(7):
        pieces = []
        for L in range(3):
            off = L * 28 + 7 * d - d * (d - 1) // 2
            sl = db[:, off:off + 7 - d]
            if d:
                sl = jnp.concatenate([sl, jnp.zeros((BA, d), f32)], axis=1)
            pieces += [sl] * _BLOCKS[L][1]
        A = jnp.concatenate(pieces + [jnp.zeros((BA, 5), f32)], axis=1)
        if d == 0:
            out = out + 2.0 * A * s
        else:
            out = out + A * _shl(s, d, 96) + _shr(A * s, d, 96)
    sb_ref[...] = jnp.concatenate([out, jnp.zeros((BA, 32), f32)], axis=1)


def _oc_fwd_body(den_ref, sp_ref, params_ref, W0, b0, g0, a0, W1, b1, a1,
                 Wout, bout, out_ref):
    masks = _species_masks(sp_ref[...])
    y, _ = _mlp_core(den_ref[...], masks, W0, b0, g0, a0, W1, b1, a1,
                     Wout=Wout, bout=bout)
    c0 = _sel(masks, params_ref)
    out_ref[...] = jnp.concatenate(
        [c0[:, 0:7] + y, jnp.zeros((BA, 121), f32)], axis=1)


def _oc_bwd_body(den_ref, sp_ref, cb_ref, W0, b0, g0, a0, W1, b1, a1,
                 W0T, W1T, WoutT, db_ref):
    masks = _species_masks(sp_ref[...])
    ybar = (cb_ref[0] + cb_ref[1])[:, 0:7]
    _, xb = _mlp_core(den_ref[...], masks, W0, b0, g0, a0, W1, b1, a1,
                      W0T=W0T, W1T=W1T, WoutT=WoutT, ybar=ybar, need_y=False)
    db_ref[...] = xb


def _nn_body(den_ref, sp_ref, init_ref, W0, b0, g0, a0, W1, b1, a1,
             Wout, bout, W0T, W1T, WoutT, out_ref, vsum_ref, db_ref):
    masks = _species_masks(sp_ref[...])
    y, xb = _mlp_core(den_ref[...], masks, W0, b0, g0, a0, W1, b1, a1,
                      Wout=Wout, bout=bout, W0T=W0T, W1T=W1T, WoutT=WoutT,
                      ybar=True, ybar_is_ones=True)
    o = y + init_ref[...]
    out_ref[...] = o
    db_ref[...] = xb
    i = pl.program_id(0)
    rid = lax.broadcasted_iota(i32, (BA, 1), 0) + i * BA
    m = (rid < N_ATOMS).astype(f32)

    @pl.when(i == 0)
    def _():
        vsum_ref[...] = jnp.zeros((1, 1), f32)

    vsum_ref[...] += jnp.sum(o * m, keepdims=True)


def _edge_bwd_body_base(gt_ref, c7, sg_ref, dv2_ref, dv_ref, cb_ref):
    gt = gt_ref[...]
    u = gt[0:3] * gt[4:5]
    inv_r = gt[4:5]
    radial = gt[5:12]
    drad = gt[12:19]
    sgT = jnp.transpose(sg_ref[...][:, 0:96])
    w = c7 * radial

    urows = [u[aa:aa + 1] for aa in range(3)]
    coefs = [None]
    for aa in range(3):
        coefs.append(urows[aa])
    for aa in range(3):
        for bb in range(3):
            coefs.append(urows[aa] * urows[bb])

    wbar = sgT[0:7]
    abar = [None]
    for j in range(1, 13):
        sl = sgT[7 * j:7 * j + 7]
        wbar = wbar + coefs[j] * sl
        abar.append(jnp.sum(sl * w, axis=0, keepdims=True))

    cb = wbar * radial
    radial_bar = wbar * c7
    rbar = jnp.sum(radial_bar * drad, axis=0, keepdims=True)

    ubars = []
    for aa in range(3):
        ub = abar[1 + aa]
        for bb in range(3):
            ub = ub + (abar[4 + 3 * aa + bb] + abar[4 + 3 * bb + aa]) * urows[bb]
        ubars.append(ub)
    ubar = jnp.concatenate(ubars, axis=0)
    udot = jnp.sum(ubar * u, axis=0, keepdims=True)
    dv = (ubar - udot * u) * inv_r + rbar * u
    dvp = jnp.concatenate([dv, jnp.zeros((5, BE), f32)], axis=0)
    dvr = jnp.transpose(dvp)
    if dv2_ref is not None:
        dvr = dvr + dv2_ref[...][:, 0:8]
    cbp = jnp.concatenate([cb, jnp.zeros((1, BE), f32)], axis=0)
    dv_ref[...] = jnp.concatenate([dvr, jnp.zeros((BE, 120), f32)], axis=1)
    cb_ref[...] = jnp.concatenate(
        [jnp.transpose(cbp), jnp.zeros((BE, 120), f32)], axis=1)


def _edge_bwd_body(gt_ref, cn_ref, sg_ref, dv_ref, cb_ref):
    c7 = jnp.transpose(cn_ref[...][:, 0:8])[0:7]
    _edge_bwd_body_base(gt_ref, c7, sg_ref, None, dv_ref, cb_ref)


def _edge_bwd_add_sp_body(gt_ref, nsp_ref, par_ref, sg_ref, dv2_ref,
                          dv_ref, cb_ref):
    spT = jnp.transpose(nsp_ref[...].astype(f32))
    c7 = _species_rows(spT, par_ref)[0:7]
    _edge_bwd_body_base(gt_ref, c7, sg_ref, dv2_ref, dv_ref, cb_ref)


def _forces_body(pn_ref, pc_ref, f_ref):
    f_ref[...] = ((pc_ref[0] + pc_ref[1]) - (pn_ref[0] + pn_ref[1]))[:, 0:16]


def _espec(D):
    return pl.BlockSpec((BE, D), lambda i: (i, 0))


def _gtspec():
    return pl.BlockSpec((32, BE), lambda i: (0, i))


def _aspec(D):
    return pl.BlockSpec((BA, D), lambda i: (i, 0))


def _pspec(D):
    return pl.BlockSpec((2, BA, D), lambda i: (0, i, 0))


def _fullspec(arr):
    nd = arr.ndim
    return pl.BlockSpec(arr.shape, lambda i, _n=nd: (0,) * _n)


_EGRID = (EP // BE,)
_AGRID = (NP // BA,)


def _tc_geom_orb1(cc, cn, nsp, rs8, inta8, params8):
    return pl.pallas_call(
        _geom_orb1_body, grid=_EGRID,
        in_specs=[_espec(128), _espec(128), _espec(1), _fullspec(rs8),
                  _fullspec(inta8), _fullspec(params8)],
        out_specs=[_gtspec(), _espec(128)],
        out_shape=[jax.ShapeDtypeStruct((32, EP), f32),
                   jax.ShapeDtypeStruct((EP, 128), f32)],
    )(cc, cn, nsp, rs8, inta8, params8)


def _tc_orb(G, Cn):
    return pl.pallas_call(
        _orb_body, grid=_EGRID,
        in_specs=[_gtspec(), _espec(128)],
        out_specs=_espec(128),
        out_shape=jax.ShapeDtypeStruct((EP, 128), f32),
    )(G, Cn)


def _tc_gram_fwd(parts):
    return pl.pallas_call(
        _gram_fwd_body, grid=_AGRID,
        in_specs=[_pspec(128)],
        out_specs=[_aspec(84), _aspec(96)],
        out_shape=[jax.ShapeDtypeStruct((NP, 84), f32),
                   jax.ShapeDtypeStruct((NP, 96), f32)],
    )(parts)


def _tc_gram_bwd(db, summed):
    return pl.pallas_call(
        _gram_bwd_body, grid=_AGRID,
        in_specs=[_aspec(84), _aspec(96)],
        out_specs=_aspec(128),
        out_shape=jax.ShapeDtypeStruct((NP, 128), f32),
    )(db, summed)


def _tc_oc_fwd(den, sp, params8, p):
    args = (den, sp, params8, p["W0"], p["b0"], p["g0"], p["a0"],
            p["W1"], p["b1"], p["a1"], p["Wout"], p["bout"])
    return pl.pallas_call(
        _oc_fwd_body, grid=_AGRID,
        in_specs=[_aspec(84), _aspec(1)] + [_fullspec(a) for a in args[2:]],
        out_specs=_aspec(128),
        out_shape=jax.ShapeDtypeStruct((NP, 128), f32),
    )(*args)


def _tc_oc_bwd(den, sp, cbparts, p):
    args = (den, sp, cbparts, p["W0"], p["b0"], p["g0"], p["a0"],
            p["W1"], p["b1"], p["a1"], p["W0T"], p["W1T"], p["WoutT"])
    return pl.pallas_call(
        _oc_bwd_body, grid=_AGRID,
        in_specs=[_aspec(84), _aspec(1), _pspec(128)] + [_fullspec(a) for a in args[3:]],
        out_specs=_aspec(84),
        out_shape=jax.ShapeDtypeStruct((NP, 84), f32),
    )(*args)


def _tc_nn(den, sp, init11, p):
    args = (den, sp, init11, p["W0"], p["b0"], p["g0"], p["a0"],
            p["W1"], p["b1"], p["a1"], p["Wout"], p["bout"],
            p["W0T"], p["W1T"], p["WoutT"])
    return pl.pallas_call(
        _nn_body, grid=_AGRID,
        in_specs=[_aspec(84), _aspec(1), _fullspec(init11)]
                 + [_fullspec(a) for a in args[3:]],
        out_specs=[_aspec(1), pl.BlockSpec((1, 1), lambda i: (0, 0)), _aspec(84)],
        out_shape=[jax.ShapeDtypeStruct((NP, 1), f32),
                   jax.ShapeDtypeStruct((1, 1), f32),
                   jax.ShapeDtypeStruct((NP, 84), f32)],
    )(*args)


def _tc_edge_bwd(G, Cn, sg):
    outs = [jax.ShapeDtypeStruct((EP, 128), f32), jax.ShapeDtypeStruct((EP, 128), f32)]
    ospecs = [_espec(128), _espec(128)]
    return pl.pallas_call(
        _edge_bwd_body, grid=_EGRID,
        in_specs=[_gtspec(), _espec(128), _espec(128)],
        out_specs=ospecs, out_shape=outs,
    )(G, Cn, sg)


def _tc_edge_bwd_sp(G, nsp, params8, sg, dv2):
    outs = [jax.ShapeDtypeStruct((EP, 128), f32), jax.ShapeDtypeStruct((EP, 128), f32)]
    ospecs = [_espec(128), _espec(128)]
    return pl.pallas_call(
        _edge_bwd_add_sp_body, grid=_EGRID,
        in_specs=[_gtspec(), _espec(1), _fullspec(params8), _espec(128),
                  _espec(128)],
        out_specs=ospecs, out_shape=outs,
    )(G, nsp, params8, sg, dv2)


def _tc_forces(pn, pc):
    return pl.pallas_call(
        _forces_body, grid=_AGRID,
        in_specs=[_pspec(128), _pspec(128)],
        out_specs=_aspec(16),
        out_shape=jax.ShapeDtypeStruct((NP, 16), f32),
    )(pn, pc)


def _prep_mlp(p, nout):
    W0 = p["W0"][:, _REFIDX, :]
    out = {
        "W0": W0,
        "b0": p["b0"], "g0": p["g0"], "a0": p["a0"],
        "W1": p["W1"], "b1": p["b1"], "a1": p["a1"],
        "Wout": p["Wout"], "bout": p["bout"],
        "W0T": jnp.swapaxes(W0, 1, 2),
        "W1T": jnp.swapaxes(p["W1"], 1, 2),
        "WoutT": jnp.swapaxes(p["Wout"], 1, 2),
    }
    return out


def kernel(cart, rs, inta, params, oc_params, nn_params, initpot,
           atom_index, local_species, neigh_species):
    center = atom_index[0].astype(i32)
    neigh = atom_index[1].astype(i32)
    pad_idx = jnp.full((EP - N_EDGES,), DUMP, i32)
    center_p = jnp.concatenate([center, pad_idx])
    neigh_p = jnp.concatenate([neigh, pad_idx])
    nsp_p = jnp.concatenate(
        [neigh_species.astype(i32), jnp.zeros((EP - N_EDGES,), i32)]).reshape(EP, 1)
    sp_p = jnp.concatenate(
        [local_species.astype(i32), jnp.zeros((NP - N_ATOMS,), i32)]).reshape(NP, 1)
    cart128 = jnp.zeros((NP, 128), f32).at[:N_ATOMS, 0:3].set(cart)
    rs8 = jnp.zeros((NTYPE, 8), f32).at[:, 0:7].set(rs)
    inta8 = jnp.zeros((NTYPE, 8), f32).at[:, 0:7].set(inta)
    params8 = jnp.zeros((NTYPE, 8), f32).at[:, 0:7].set(params)
    ocP = _prep_mlp(oc_params, NWAVE)
    nnP = _prep_mlp(nn_params, 1)
    init11 = jnp.reshape(initpot, (1, 1)).astype(f32)

    cc = _sc_gather(cart128, center_p, 128)
    cn = _sc_gather(cart128, neigh_p, 128)
    G, orb1 = _tc_geom_orb1(cc, cn, nsp_p, rs8, inta8, params8)
    parts1 = _sc_scatter_add(orb1, center_p, 128)
    den1, sum1 = _tc_gram_fwd(parts1)

    C1p = _tc_oc_fwd(den1, sp_p, params8, ocP)
    Cn2 = _sc_gather(C1p, neigh_p, 128)
    orb2 = _tc_orb(G, Cn2)
    parts2 = _sc_scatter_add(orb2, center_p, 128)
    den2, sum2 = _tc_gram_fwd(parts2)

    out_full, vsum, dbar2 = _tc_nn(den2, sp_p, init11, nnP)

    sbar2 = _tc_gram_bwd(dbar2, sum2)
    sg2 = _sc_gather(sbar2, center_p, 128)
    dv2, cb2 = _tc_edge_bwd(G, Cn2, sg2)
    cbparts = _sc_scatter_add(cb2, neigh_p, 128)

    dbar1 = _tc_oc_bwd(den1, sp_p, cbparts, ocP)

    sbar1 = _tc_gram_bwd(dbar1, sum1)
    sg1 = _sc_gather(sbar1, center_p, 128)
    dvt, _ = _tc_edge_bwd_sp(G, nsp_p, params8, sg1, dv2)

    pn = _sc_scatter_add(dvt, neigh_p, 128)
    pc = _sc_scatter_add(dvt, center_p, 128)
    F = _tc_forces(pn, pc)

    varene = vsum[0, 0]
    forces = F[:N_ATOMS, 0:3].reshape(-1)
    output = out_full[:N_ATOMS]
    return varene, forces, output
